# baseline (speedup 1.0000x reference)
import functools

import jax
import jax.numpy as jnp
from jax import lax
from jax.experimental import pallas as pl
from jax.experimental.pallas import tpu as pltpu
from jax.experimental.pallas import tpu_sc as plsc

NC = 2
NS = 16
NW = NC * NS

CHUNK = 128


def _seg_sum_sc(embeddings, adj_src, adj_dst, zeros):
    N, D = embeddings.shape
    R, NCHT, _ = adj_src.shape
    rel_per_core = R // NC
    npt = (N // NS) // 8 * 8
    npt_last = N - (NS - 1) * npt
    NCH = (-(-NCHT // NS) + 7) // 8 * 8
    NCH_LAST = NCHT - (NS - 1) * NCH
    assert 0 < NCH_LAST <= NCH
    NBUF = 2

    mesh = plsc.VectorSubcoreMesh(core_axis_name="c", subcore_axis_name="s")

    @functools.partial(
        pl.kernel,
        out_type=jax.ShapeDtypeStruct((R, N, D), jnp.float32),
        mesh=mesh,
        scratch_types=[
            pltpu.VMEM((NCH, CHUNK), jnp.int32),
            pltpu.VMEM((NCH, CHUNK), jnp.int32),
            pltpu.VMEM((NBUF, CHUNK, D), jnp.float32),
            pltpu.VMEM_SHARED((N, D), jnp.float32),
            pltpu.SemaphoreType.DMA,
            pltpu.SemaphoreType.DMA,
        ],
    )
    def seg_sum(emb, asrc, adst, zer, out, src_i, dst_i, rows, acc,
                sem_g, sem_s):
        c = lax.axis_index("c")
        s = lax.axis_index("s")

        def fire_gather(j):
            pltpu.async_copy(emb.at[src_i.at[j]],
                             rows.at[lax.rem(j, NBUF)], sem_g)

        def wait_gather():
            pltpu.make_async_copy(emb.at[src_i.at[0]],
                                  rows.at[0], sem_g).wait()

        def wait_scatter():
            pltpu.make_async_copy(rows.at[0],
                                  acc.at[pl.ds(0, CHUNK)], sem_s).wait()

        for rloc in range(rel_per_core):
            r = c * rel_per_core + rloc
            @pl.when(s < NS - 1)
            def _():
                pltpu.sync_copy(zer.at[pl.ds(0, npt)],
                                acc.at[pl.ds(s * npt, npt)])

            @pl.when(s == NS - 1)
            def _():
                pltpu.sync_copy(zer.at[pl.ds(0, npt_last)],
                                acc.at[pl.ds((NS - 1) * npt, npt_last)])

            @pl.when(s < NS - 1)
            def _():
                pltpu.sync_copy(asrc.at[r, pl.ds(s * NCH, NCH)],
                                src_i.at[pl.ds(0, NCH)])
                pltpu.sync_copy(adst.at[r, pl.ds(s * NCH, NCH)],
                                dst_i.at[pl.ds(0, NCH)])

            @pl.when(s == NS - 1)
            def _():
                pltpu.sync_copy(
                    asrc.at[r, pl.ds((NS - 1) * NCH, NCH_LAST)],
                    src_i.at[pl.ds(0, NCH_LAST)])
                pltpu.sync_copy(
                    adst.at[r, pl.ds((NS - 1) * NCH, NCH_LAST)],
                    dst_i.at[pl.ds(0, NCH_LAST)])

            nch_t = jnp.where(s < NS - 1, NCH, NCH_LAST)

            for k in range(NBUF - 1):
                fire_gather(k)

            plsc.subcore_barrier()

            @pl.loop(0, nch_t)
            def _(j):
                wait_gather()

                @pl.when(j >= 1)
                def _():
                    wait_scatter()

                @pl.when(j + NBUF - 1 < nch_t)
                def _():
                    fire_gather(j + NBUF - 1)

                pltpu.async_copy(rows.at[lax.rem(j, NBUF)],
                                 acc.at[dst_i.at[j]], sem_s, add=True)

            wait_scatter()
            plsc.subcore_barrier()

            @pl.when(s < NS - 1)
            def _():
                pltpu.sync_copy(acc.at[pl.ds(s * npt, npt)],
                                out.at[r, pl.ds(s * npt, npt)])

            @pl.when(s == NS - 1)
            def _():
                pltpu.sync_copy(acc.at[pl.ds((NS - 1) * npt, npt_last)],
                                out.at[r, pl.ds((NS - 1) * npt, npt_last)])

            plsc.subcore_barrier()

    return seg_sum(embeddings, adj_src, adj_dst, zeros)


def _gather_sc(node_out, ht_idx):
    N, D = node_out.shape
    ncht = ht_idx.shape[0]
    B = ncht * CHUNK // 2
    nch = ncht // NW
    assert nch * NW == ncht and nch % 8 == 0

    mesh = plsc.VectorSubcoreMesh(core_axis_name="c", subcore_axis_name="s")

    @functools.partial(
        pl.kernel,
        out_type=(jax.ShapeDtypeStruct((B, D), jnp.float32),
                  jax.ShapeDtypeStruct((B, D), jnp.float32)),
        mesh=mesh,
        scratch_types=[
            pltpu.VMEM((nch, CHUNK), jnp.int32),
            pltpu.VMEM((4, CHUNK, D), jnp.float32),
            pltpu.SemaphoreType.DMA,
            pltpu.SemaphoreType.DMA,
        ],
    )
    def gat(node, htidx, oh, ot, idx_v, rows, sem, sem_st):
        c = lax.axis_index("c")
        s = lax.axis_index("s")
        w = s * NC + c

        pltpu.sync_copy(htidx.at[pl.ds(w * nch, nch)], idx_v)
        NB = 4
        gd = [None] * nch
        std = [None] * nch

        def store(j):
            def do(dst, off):
                return pltpu.async_copy(
                    rows.at[j % NB], dst.at[pl.ds(off * CHUNK, CHUNK)],
                    sem_st)

            @pl.when(w < NW // 2)
            def _():
                do(oh, w * nch + j)

            @pl.when(w >= NW // 2)
            def _():
                do(ot, (w - NW // 2) * nch + j)
            return pltpu.make_async_copy(
                rows.at[j % NB], oh.at[pl.ds(0, CHUNK)], sem_st)

        for k in range(min(NB - 1, nch)):
            gd[k] = pltpu.async_copy(node.at[idx_v.at[k]], rows.at[k], sem)
        for j in range(nch):
            gd[j].wait()
            std[j] = store(j)
            nxt = j + NB - 1
            if nxt < nch:
                if j >= 1:
                    std[j - 1].wait()
                    std[j - 1] = None
                gd[nxt] = pltpu.async_copy(
                    node.at[idx_v.at[nxt]], rows.at[nxt % NB], sem)
        for j in range(nch):
            if std[j] is not None:
                std[j].wait()

    return gat(node_out, ht_idx)


def _relation_matmul_tc(S, relation_kernel, acc_in=None):
    R, N, D = S.shape
    OUT = relation_kernel.shape[-1]
    BN = 2000

    def mm(*refs):
        if acc_in is None:
            s_ref, w_ref, o_ref = refs
            acc = jnp.zeros((BN, OUT), jnp.float32)
        else:
            s_ref, w_ref, a_ref, o_ref = refs
            acc = a_ref[...]
        for r in range(R):
            acc += jnp.dot(s_ref[r], w_ref[r],
                           preferred_element_type=jnp.float32)
        o_ref[...] = acc

    in_specs = [
        pl.BlockSpec((R, BN, D), lambda i: (0, i, 0)),
        pl.BlockSpec((R, D, OUT), lambda i: (0, 0, 0)),
    ]
    args = [S, relation_kernel]
    if acc_in is not None:
        in_specs.append(pl.BlockSpec((BN, OUT), lambda i: (i, 0)))
        args.append(acc_in)

    return pl.pallas_call(
        mm,
        grid=(N // BN,),
        in_specs=in_specs,
        out_specs=pl.BlockSpec((BN, OUT), lambda i: (i, 0)),
        out_shape=jax.ShapeDtypeStruct((N, OUT), jnp.float32),
    )(*args)


def _base_tc(head_e, tail_e, self_kernel):
    B, D = head_e.shape
    OUT = self_kernel.shape[-1]
    BB = 2048

    def mm(he, te, sk, oh, ot):
        oh[...] = jnp.dot(he[...], sk[...],
                          preferred_element_type=jnp.float32)
        ot[...] = jnp.dot(te[...], sk[...],
                          preferred_element_type=jnp.float32)

    return pl.pallas_call(
        mm,
        grid=(B // BB,),
        in_specs=[
            pl.BlockSpec((BB, D), lambda i: (i, 0)),
            pl.BlockSpec((BB, D), lambda i: (i, 0)),
            pl.BlockSpec((D, OUT), lambda i: (0, 0)),
        ],
        out_specs=(pl.BlockSpec((BB, OUT), lambda i: (i, 0)),
                   pl.BlockSpec((BB, OUT), lambda i: (i, 0))),
        out_shape=(jax.ShapeDtypeStruct((B, OUT), jnp.float32),
                   jax.ShapeDtypeStruct((B, OUT), jnp.float32)),
    )(head_e, tail_e, self_kernel)


def _final_tc(base_h, base_t, gath_h, gath_t):
    B, OUT = base_h.shape
    BB = 2048

    def fin(bh, bt, gh, gt, oh, ot):
        oh[...] = jax.nn.sigmoid(bh[...] + gh[...])
        ot[...] = jax.nn.sigmoid(bt[...] + gt[...])

    spec = pl.BlockSpec((BB, OUT), lambda i: (i, 0))
    return pl.pallas_call(
        fin,
        grid=(B // BB,),
        in_specs=[spec] * 4,
        out_specs=(spec, spec),
        out_shape=(jax.ShapeDtypeStruct((B, OUT), jnp.float32),
                   jax.ShapeDtypeStruct((B, OUT), jnp.float32)),
    )(base_h, base_t, gath_h, gath_t)


def kernel(embeddings, head_idx, head_e, tail_idx, tail_e, adj_src, adj_dst,
           relation_kernel, self_kernel):
    N, D = embeddings.shape
    R, E = adj_src.shape
    zeros = jnp.zeros((N - (NS - 1) * ((N // NS) // 8 * 8), D), jnp.float32)
    asrc = adj_src.reshape(R, E // CHUNK, CHUNK)
    adst = adj_dst.reshape(R, E // CHUNK, CHUNK)
    half = R // 2
    base_h, base_t = _base_tc(head_e, tail_e, self_kernel)
    S_a = _seg_sum_sc(embeddings, asrc[:half], adst[:half], zeros)
    S_b = _seg_sum_sc(embeddings, asrc[half:], adst[half:], zeros)
    node_out = _relation_matmul_tc(S_a, relation_kernel[:half])
    node_out = _relation_matmul_tc(S_b, relation_kernel[half:], node_out)
    ht_idx = jnp.concatenate([head_idx, tail_idx]).reshape(-1, CHUNK)
    gath_h, gath_t = _gather_sc(node_out, ht_idx)
    return _final_tc(base_h, base_t, gath_h, gath_t)


# speedup vs baseline: 1.2486x; 1.2486x over previous
import functools

import jax
import jax.numpy as jnp
from jax import lax
from jax.experimental import pallas as pl
from jax.experimental.pallas import tpu as pltpu
from jax.experimental.pallas import tpu_sc as plsc

NC = 2
NS = 16
NW = NC * NS

CHUNK = 128


def _seg_sum_sc(embeddings, adj_pair, zeros):
    N, D = embeddings.shape
    R, NCHT = adj_pair.shape[:2]
    rel_per_core = R // NC
    npt = (N // NS) // 8 * 8
    npt_last = N - (NS - 1) * npt
    NCH = -(-NCHT // NS)
    NCH_LAST = NCHT - (NS - 1) * NCH
    assert 0 < NCH_LAST <= NCH
    NRB = 3
    NIB = 4

    mesh = plsc.VectorSubcoreMesh(core_axis_name="c", subcore_axis_name="s")

    @functools.partial(
        pl.kernel,
        out_type=jax.ShapeDtypeStruct((R, N, D), jnp.float32),
        mesh=mesh,
        scratch_types=[
            pltpu.VMEM((NIB, 2, CHUNK), jnp.int32),
            pltpu.VMEM((NRB, CHUNK, D), jnp.float32),
            pltpu.VMEM_SHARED((N, D), jnp.float32),
            pltpu.SemaphoreType.DMA,
            pltpu.SemaphoreType.DMA,
            pltpu.SemaphoreType.DMA,
        ],
    )
    def seg_sum(emb, adj, zer, out, idx, rows, acc, sem_i, sem_g, sem_s):
        c = lax.axis_index("c")
        s = lax.axis_index("s")

        def fire_idx(r, base, j):
            pltpu.async_copy(adj.at[r, base + j], idx.at[lax.rem(j, NIB)],
                             sem_i)

        def wait_idx():
            pltpu.make_async_copy(adj.at[0, 0], idx.at[0], sem_i).wait()

        def fire_gather(j):
            pltpu.async_copy(emb.at[idx.at[lax.rem(j, NIB), 0]],
                             rows.at[lax.rem(j, NRB)], sem_g)

        def wait_gather():
            pltpu.make_async_copy(emb.at[idx.at[0, 0]], rows.at[0],
                                  sem_g).wait()

        def fire_scatter(j):
            pltpu.async_copy(rows.at[lax.rem(j, NRB)],
                             acc.at[idx.at[lax.rem(j, NIB), 1]], sem_s,
                             add=True)

        def wait_scatter():
            pltpu.make_async_copy(rows.at[0], acc.at[pl.ds(0, CHUNK)],
                                  sem_s).wait()

        for rloc in range(rel_per_core):
            r = c * rel_per_core + rloc
            base = s * NCH
            nch_t = jnp.where(s < NS - 1, NCH, NCH_LAST)

            fire_idx(r, base, 0)
            fire_idx(r, base, 1)

            @pl.when(s < NS - 1)
            def _():
                pltpu.sync_copy(zer.at[pl.ds(0, npt)],
                                acc.at[pl.ds(s * npt, npt)])

            @pl.when(s == NS - 1)
            def _():
                pltpu.sync_copy(zer.at[pl.ds(0, npt_last)],
                                acc.at[pl.ds((NS - 1) * npt, npt_last)])

            wait_idx()
            fire_gather(0)
            plsc.subcore_barrier()

            @pl.loop(0, nch_t)
            def _(j):
                @pl.when(j >= 2)
                def _():
                    wait_scatter()

                @pl.when(j + 2 < nch_t)
                def _():
                    fire_idx(r, base, j + 2)

                @pl.when(j + 1 < nch_t)
                def _():
                    wait_idx()
                    fire_gather(j + 1)

                wait_gather()
                fire_scatter(j)

            wait_scatter()

            @pl.when(nch_t >= 2)
            def _():
                wait_scatter()

            plsc.subcore_barrier()

            @pl.when(s < NS - 1)
            def _():
                pltpu.sync_copy(acc.at[pl.ds(s * npt, npt)],
                                out.at[r, pl.ds(s * npt, npt)])

            @pl.when(s == NS - 1)
            def _():
                pltpu.sync_copy(acc.at[pl.ds((NS - 1) * npt, npt_last)],
                                out.at[r, pl.ds((NS - 1) * npt, npt_last)])

            plsc.subcore_barrier()

    return seg_sum(embeddings, adj_pair, zeros)


def _gather_sc(node_out, ht_idx):
    N, D = node_out.shape
    ncht = ht_idx.shape[0]
    B = ncht * CHUNK // 2
    nch = ncht // NW
    assert nch * NW == ncht and nch % 8 == 0

    mesh = plsc.VectorSubcoreMesh(core_axis_name="c", subcore_axis_name="s")

    @functools.partial(
        pl.kernel,
        out_type=(jax.ShapeDtypeStruct((B, D), jnp.float32),
                  jax.ShapeDtypeStruct((B, D), jnp.float32)),
        mesh=mesh,
        scratch_types=[
            pltpu.VMEM((nch, CHUNK), jnp.int32),
            pltpu.VMEM((2, CHUNK, D), jnp.float32),
            pltpu.SemaphoreType.DMA,
        ],
    )
    def gat(node, htidx, oh, ot, idx_v, rows, sem):
        c = lax.axis_index("c")
        s = lax.axis_index("s")
        w = s * NC + c

        pltpu.sync_copy(htidx.at[pl.ds(w * nch, nch)], idx_v)
        descs = [None] * nch
        descs[0] = pltpu.async_copy(node.at[idx_v.at[0]], rows.at[0], sem)
        for j in range(nch):
            if j + 1 < nch:
                descs[j + 1] = pltpu.async_copy(
                    node.at[idx_v.at[j + 1]], rows.at[(j + 1) % 2], sem)
            descs[j].wait()

            @pl.when(w < NW // 2)
            def _():
                pltpu.sync_copy(
                    rows.at[j % 2],
                    oh.at[pl.ds((w * nch + j) * CHUNK, CHUNK)])

            @pl.when(w >= NW // 2)
            def _():
                pltpu.sync_copy(
                    rows.at[j % 2],
                    ot.at[pl.ds(((w - NW // 2) * nch + j) * CHUNK, CHUNK)])

    return gat(node_out, ht_idx)


def _relation_matmul_tc(S, relation_kernel):
    R, N, D = S.shape
    OUT = relation_kernel.shape[-1]
    BN = 2000

    def mm(s_ref, w_ref, o_ref):
        acc = jnp.dot(s_ref[0], w_ref[0], preferred_element_type=jnp.float32)
        for r in range(1, R):
            acc += jnp.dot(s_ref[r], w_ref[r],
                           preferred_element_type=jnp.float32)
        o_ref[...] = acc

    return pl.pallas_call(
        mm,
        grid=(N // BN,),
        in_specs=[
            pl.BlockSpec((R, BN, D), lambda i: (0, i, 0)),
            pl.BlockSpec((R, D, OUT), lambda i: (0, 0, 0)),
        ],
        out_specs=pl.BlockSpec((BN, OUT), lambda i: (i, 0)),
        out_shape=jax.ShapeDtypeStruct((N, OUT), jnp.float32),
    )(S, relation_kernel)


def _final_tc(head_e, tail_e, gath_h, gath_t, self_kernel):
    B, D = head_e.shape
    OUT = self_kernel.shape[-1]
    BB = 2048

    def fin(he, te, gh, gt, sk, oh, ot):
        oh[...] = jax.nn.sigmoid(
            jnp.dot(he[...], sk[...], preferred_element_type=jnp.float32)
            + gh[...])
        ot[...] = jax.nn.sigmoid(
            jnp.dot(te[...], sk[...], preferred_element_type=jnp.float32)
            + gt[...])

    return pl.pallas_call(
        fin,
        grid=(B // BB,),
        in_specs=[
            pl.BlockSpec((BB, D), lambda i: (i, 0)),
            pl.BlockSpec((BB, D), lambda i: (i, 0)),
            pl.BlockSpec((BB, OUT), lambda i: (i, 0)),
            pl.BlockSpec((BB, OUT), lambda i: (i, 0)),
            pl.BlockSpec((D, OUT), lambda i: (0, 0)),
        ],
        out_specs=(pl.BlockSpec((BB, OUT), lambda i: (i, 0)),
                   pl.BlockSpec((BB, OUT), lambda i: (i, 0))),
        out_shape=(jax.ShapeDtypeStruct((B, OUT), jnp.float32),
                   jax.ShapeDtypeStruct((B, OUT), jnp.float32)),
    )(head_e, tail_e, gath_h, gath_t, self_kernel)


def kernel(embeddings, head_idx, head_e, tail_idx, tail_e, adj_src, adj_dst,
           relation_kernel, self_kernel):
    N, D = embeddings.shape
    R, E = adj_src.shape
    zeros = jnp.zeros((N - (NS - 1) * ((N // NS) // 8 * 8), D), jnp.float32)
    adj_pair = jnp.stack([adj_src.reshape(R, E // CHUNK, CHUNK),
                          adj_dst.reshape(R, E // CHUNK, CHUNK)], axis=2)
    S = _seg_sum_sc(embeddings, adj_pair, zeros)
    node_out = _relation_matmul_tc(S, relation_kernel)
    ht_idx = jnp.concatenate([head_idx, tail_idx]).reshape(-1, CHUNK)
    gath_h, gath_t = _gather_sc(node_out, ht_idx)
    return _final_tc(head_e, tail_e, gath_h, gath_t, self_kernel)


# speedup vs baseline: 1.2510x; 1.0019x over previous
import functools

import jax
import jax.numpy as jnp
from jax import lax
from jax.experimental import pallas as pl
from jax.experimental.pallas import tpu as pltpu
from jax.experimental.pallas import tpu_sc as plsc

NC = 2
NS = 16
NW = NC * NS

CHUNK = 128


def _seg_sum_sc(embeddings, adj_pair, zeros):
    N, D = embeddings.shape
    R, NCHT = adj_pair.shape[:2]
    rel_per_core = R // NC
    npt = (N // NS) // 8 * 8
    npt_last = N - (NS - 1) * npt
    NCH = -(-NCHT // NS)
    NCH_LAST = NCHT - (NS - 1) * NCH
    assert 0 < NCH_LAST <= NCH
    NRB = 3
    NIB = 4

    mesh = plsc.VectorSubcoreMesh(core_axis_name="c", subcore_axis_name="s")

    @functools.partial(
        pl.kernel,
        out_type=jax.ShapeDtypeStruct((R, N, D), jnp.float32),
        mesh=mesh,
        scratch_types=[
            pltpu.VMEM((NIB, 2, CHUNK), jnp.int32),
            pltpu.VMEM((NRB, CHUNK, D), jnp.float32),
            pltpu.VMEM_SHARED((N, D), jnp.float32),
            pltpu.SemaphoreType.DMA,
            pltpu.SemaphoreType.DMA,
            pltpu.SemaphoreType.DMA,
        ],
    )
    def seg_sum(emb, adj, zer, out, idx, rows, acc, sem_i, sem_g, sem_s):
        c = lax.axis_index("c")
        s = lax.axis_index("s")

        def fire_idx(r, base, j):
            pltpu.async_copy(adj.at[r, base + j], idx.at[lax.rem(j, NIB)],
                             sem_i)

        def wait_idx():
            pltpu.make_async_copy(adj.at[0, 0], idx.at[0], sem_i).wait()

        def fire_gather(j):
            pltpu.async_copy(emb.at[idx.at[lax.rem(j, NIB), 0]],
                             rows.at[lax.rem(j, NRB)], sem_g)

        def wait_gather():
            pltpu.make_async_copy(emb.at[idx.at[0, 0]], rows.at[0],
                                  sem_g).wait()

        def fire_scatter(j):
            pltpu.async_copy(rows.at[lax.rem(j, NRB)],
                             acc.at[idx.at[lax.rem(j, NIB), 1]], sem_s,
                             add=True)

        def wait_scatter():
            pltpu.make_async_copy(rows.at[0], acc.at[pl.ds(0, CHUNK)],
                                  sem_s).wait()

        for rloc in range(rel_per_core):
            r = c * rel_per_core + rloc
            base = s * NCH
            nch_t = jnp.where(s < NS - 1, NCH, NCH_LAST)

            fire_idx(r, base, 0)
            fire_idx(r, base, 1)

            @pl.when(s < NS - 1)
            def _():
                pltpu.sync_copy(zer.at[pl.ds(0, npt)],
                                acc.at[pl.ds(s * npt, npt)])

            @pl.when(s == NS - 1)
            def _():
                pltpu.sync_copy(zer.at[pl.ds(0, npt_last)],
                                acc.at[pl.ds((NS - 1) * npt, npt_last)])

            wait_idx()
            fire_gather(0)
            plsc.subcore_barrier()

            @pl.loop(0, nch_t)
            def _(j):
                @pl.when(j >= 2)
                def _():
                    wait_scatter()

                @pl.when(j + 2 < nch_t)
                def _():
                    fire_idx(r, base, j + 2)

                @pl.when(j + 1 < nch_t)
                def _():
                    wait_idx()
                    fire_gather(j + 1)

                wait_gather()
                fire_scatter(j)

            wait_scatter()

            @pl.when(nch_t >= 2)
            def _():
                wait_scatter()

            plsc.subcore_barrier()

            @pl.when(s < NS - 1)
            def _():
                pltpu.sync_copy(acc.at[pl.ds(s * npt, npt)],
                                out.at[r, pl.ds(s * npt, npt)])

            @pl.when(s == NS - 1)
            def _():
                pltpu.sync_copy(acc.at[pl.ds((NS - 1) * npt, npt_last)],
                                out.at[r, pl.ds((NS - 1) * npt, npt_last)])

            plsc.subcore_barrier()

    return seg_sum(embeddings, adj_pair, zeros)


def _gather_sc(node_out, ht_idx):
    N, D = node_out.shape
    ncht = ht_idx.shape[0]
    B = ncht * CHUNK // 2
    nch = ncht // NW
    assert nch * NW == ncht and nch % 8 == 0

    mesh = plsc.VectorSubcoreMesh(core_axis_name="c", subcore_axis_name="s")
    NB = 4

    @functools.partial(
        pl.kernel,
        out_type=jax.ShapeDtypeStruct((2 * B, D), jnp.float32),
        mesh=mesh,
        scratch_types=[
            pltpu.VMEM((nch, CHUNK), jnp.int32),
            pltpu.VMEM((NB, CHUNK, D), jnp.float32),
            pltpu.SemaphoreType.DMA,
            pltpu.SemaphoreType.DMA,
        ],
    )
    def gat(node, htidx, o, idx_v, rows, sem, sem_st):
        c = lax.axis_index("c")
        s = lax.axis_index("s")
        w = s * NC + c

        pltpu.sync_copy(htidx.at[pl.ds(w * nch, nch)], idx_v)
        gd = [None] * nch
        std = [None] * nch
        for k in range(min(NB - 1, nch)):
            gd[k] = pltpu.async_copy(node.at[idx_v.at[k]], rows.at[k], sem)
        for j in range(nch):
            gd[j].wait()
            std[j] = pltpu.async_copy(
                rows.at[j % NB],
                o.at[pl.ds((w * nch + j) * CHUNK, CHUNK)], sem_st)
            nxt = j + NB - 1
            if nxt < nch:
                if j >= 1:
                    std[j - 1].wait()
                    std[j - 1] = None
                gd[nxt] = pltpu.async_copy(
                    node.at[idx_v.at[nxt]], rows.at[nxt % NB], sem)
        for j in range(nch):
            if std[j] is not None:
                std[j].wait()

    return gat(node_out, ht_idx)


def _relation_matmul_tc(S, relation_kernel):
    R, N, D = S.shape
    OUT = relation_kernel.shape[-1]
    BN = 2000

    def mm(s_ref, w_ref, o_ref):
        acc = jnp.dot(s_ref[0], w_ref[0], preferred_element_type=jnp.float32)
        for r in range(1, R):
            acc += jnp.dot(s_ref[r], w_ref[r],
                           preferred_element_type=jnp.float32)
        o_ref[...] = acc

    return pl.pallas_call(
        mm,
        grid=(N // BN,),
        in_specs=[
            pl.BlockSpec((R, BN, D), lambda i: (0, i, 0)),
            pl.BlockSpec((R, D, OUT), lambda i: (0, 0, 0)),
        ],
        out_specs=pl.BlockSpec((BN, OUT), lambda i: (i, 0)),
        out_shape=jax.ShapeDtypeStruct((N, OUT), jnp.float32),
    )(S, relation_kernel)


def _final_tc(head_e, tail_e, gath_ht, self_kernel):
    B, D = head_e.shape
    OUT = self_kernel.shape[-1]
    BB = 2048
    nblk = B // BB

    def fin(he, te, gh, gt, sk, oh, ot):
        oh[...] = jax.nn.sigmoid(
            jnp.dot(he[...], sk[...], preferred_element_type=jnp.float32)
            + gh[...])
        ot[...] = jax.nn.sigmoid(
            jnp.dot(te[...], sk[...], preferred_element_type=jnp.float32)
            + gt[...])

    return pl.pallas_call(
        fin,
        grid=(nblk,),
        in_specs=[
            pl.BlockSpec((BB, D), lambda i: (i, 0)),
            pl.BlockSpec((BB, D), lambda i: (i, 0)),
            pl.BlockSpec((BB, OUT), lambda i: (i, 0)),
            pl.BlockSpec((BB, OUT), lambda i: (i + nblk, 0)),
            pl.BlockSpec((D, OUT), lambda i: (0, 0)),
        ],
        out_specs=(pl.BlockSpec((BB, OUT), lambda i: (i, 0)),
                   pl.BlockSpec((BB, OUT), lambda i: (i, 0))),
        out_shape=(jax.ShapeDtypeStruct((B, OUT), jnp.float32),
                   jax.ShapeDtypeStruct((B, OUT), jnp.float32)),
    )(head_e, tail_e, gath_ht, gath_ht, self_kernel)


def kernel(embeddings, head_idx, head_e, tail_idx, tail_e, adj_src, adj_dst,
           relation_kernel, self_kernel):
    N, D = embeddings.shape
    R, E = adj_src.shape
    zeros = jnp.zeros((N - (NS - 1) * ((N // NS) // 8 * 8), D), jnp.float32)
    adj_pair = jnp.stack([adj_src.reshape(R, E // CHUNK, CHUNK),
                          adj_dst.reshape(R, E // CHUNK, CHUNK)], axis=2)
    S = _seg_sum_sc(embeddings, adj_pair, zeros)
    node_out = _relation_matmul_tc(S, relation_kernel)
    ht_idx = jnp.concatenate([head_idx, tail_idx]).reshape(-1, CHUNK)
    gath_ht = _gather_sc(node_out, ht_idx)
    return _final_tc(head_e, tail_e, gath_ht, self_kernel)


# speedup vs baseline: 1.2750x; 1.0192x over previous
import functools

import jax
import jax.numpy as jnp
from jax import lax
from jax.experimental import pallas as pl
from jax.experimental.pallas import tpu as pltpu
from jax.experimental.pallas import tpu_sc as plsc

NC = 2
NS = 16
NW = NC * NS

CHUNK = 128


def _seg_sum_sc(embeddings, adj_src, adj_dst, zeros):
    N, D = embeddings.shape
    R, NCHT = adj_src.shape[:2]
    rel_per_core = R // NC
    npt = (N // NS) // 8 * 8
    npt_last = N - (NS - 1) * npt
    NCH = -(-NCHT // NS)
    NCH_LAST = NCHT - (NS - 1) * NCH
    assert 0 < NCH_LAST <= NCH
    NRB = 3
    NIB = 4

    mesh = plsc.VectorSubcoreMesh(core_axis_name="c", subcore_axis_name="s")

    @functools.partial(
        pl.kernel,
        out_type=jax.ShapeDtypeStruct((R, N, D), jnp.float32),
        mesh=mesh,
        scratch_types=[
            pltpu.VMEM((NIB, CHUNK), jnp.int32),
            pltpu.VMEM((NIB, CHUNK), jnp.int32),
            pltpu.VMEM((NRB, CHUNK, D), jnp.float32),
            pltpu.VMEM_SHARED((N, D), jnp.float32),
            pltpu.SemaphoreType.DMA,
            pltpu.SemaphoreType.DMA,
            pltpu.SemaphoreType.DMA,
        ],
    )
    def seg_sum(emb, asrc, adst, zer, out, isrc, idst, rows, acc,
                sem_i, sem_g, sem_s):
        c = lax.axis_index("c")
        s = lax.axis_index("s")

        def fire_idx(r, base, j):
            b = lax.rem(j, NIB)
            pltpu.async_copy(asrc.at[r, base + j], isrc.at[b], sem_i)
            pltpu.async_copy(adst.at[r, base + j], idst.at[b], sem_i)

        def wait_idx():
            pltpu.make_async_copy(asrc.at[0, 0], isrc.at[0], sem_i).wait()
            pltpu.make_async_copy(adst.at[0, 0], idst.at[0], sem_i).wait()

        def fire_gather(j):
            pltpu.async_copy(emb.at[isrc.at[lax.rem(j, NIB)]],
                             rows.at[lax.rem(j, NRB)], sem_g)

        def wait_gather():
            pltpu.make_async_copy(emb.at[isrc.at[0]], rows.at[0],
                                  sem_g).wait()

        def fire_scatter(j):
            pltpu.async_copy(rows.at[lax.rem(j, NRB)],
                             acc.at[idst.at[lax.rem(j, NIB)]], sem_s,
                             add=True)

        def wait_scatter():
            pltpu.make_async_copy(rows.at[0], acc.at[pl.ds(0, CHUNK)],
                                  sem_s).wait()

        for rloc in range(rel_per_core):
            r = c * rel_per_core + rloc
            base = s * NCH
            nch_t = jnp.where(s < NS - 1, NCH, NCH_LAST)

            fire_idx(r, base, 0)
            fire_idx(r, base, 1)

            @pl.when(s < NS - 1)
            def _():
                pltpu.sync_copy(zer.at[pl.ds(0, npt)],
                                acc.at[pl.ds(s * npt, npt)])

            @pl.when(s == NS - 1)
            def _():
                pltpu.sync_copy(zer.at[pl.ds(0, npt_last)],
                                acc.at[pl.ds((NS - 1) * npt, npt_last)])

            wait_idx()
            fire_gather(0)
            plsc.subcore_barrier()

            @pl.loop(0, nch_t)
            def _(j):
                @pl.when(j >= 2)
                def _():
                    wait_scatter()

                @pl.when(j + 2 < nch_t)
                def _():
                    fire_idx(r, base, j + 2)

                @pl.when(j + 1 < nch_t)
                def _():
                    wait_idx()
                    fire_gather(j + 1)

                wait_gather()
                fire_scatter(j)

            wait_scatter()

            @pl.when(nch_t >= 2)
            def _():
                wait_scatter()

            plsc.subcore_barrier()

            @pl.when(s < NS - 1)
            def _():
                pltpu.sync_copy(acc.at[pl.ds(s * npt, npt)],
                                out.at[r, pl.ds(s * npt, npt)])

            @pl.when(s == NS - 1)
            def _():
                pltpu.sync_copy(acc.at[pl.ds((NS - 1) * npt, npt_last)],
                                out.at[r, pl.ds((NS - 1) * npt, npt_last)])

            plsc.subcore_barrier()

    return seg_sum(embeddings, adj_src, adj_dst, zeros)


def _gather_sc(node_out, ht_idx):
    N, D = node_out.shape
    ncht = ht_idx.shape[0]
    B = ncht * CHUNK // 2
    nch = ncht // NW
    assert nch * NW == ncht and nch % 8 == 0

    mesh = plsc.VectorSubcoreMesh(core_axis_name="c", subcore_axis_name="s")
    NB = 4

    @functools.partial(
        pl.kernel,
        out_type=jax.ShapeDtypeStruct((2 * B, D), jnp.float32),
        mesh=mesh,
        scratch_types=[
            pltpu.VMEM((nch, CHUNK), jnp.int32),
            pltpu.VMEM((NB, CHUNK, D), jnp.float32),
            pltpu.SemaphoreType.DMA,
            pltpu.SemaphoreType.DMA,
        ],
    )
    def gat(node, htidx, o, idx_v, rows, sem, sem_st):
        c = lax.axis_index("c")
        s = lax.axis_index("s")
        w = s * NC + c

        pltpu.sync_copy(htidx.at[pl.ds(w * nch, nch)], idx_v)
        gd = [None] * nch
        std = [None] * nch
        for k in range(min(NB - 1, nch)):
            gd[k] = pltpu.async_copy(node.at[idx_v.at[k]], rows.at[k], sem)
        for j in range(nch):
            gd[j].wait()
            std[j] = pltpu.async_copy(
                rows.at[j % NB],
                o.at[pl.ds((w * nch + j) * CHUNK, CHUNK)], sem_st)
            nxt = j + NB - 1
            if nxt < nch:
                if j >= 1:
                    std[j - 1].wait()
                    std[j - 1] = None
                gd[nxt] = pltpu.async_copy(
                    node.at[idx_v.at[nxt]], rows.at[nxt % NB], sem)
        for j in range(nch):
            if std[j] is not None:
                std[j].wait()

    return gat(node_out, ht_idx)


def _relation_matmul_tc(S, relation_kernel):
    R, N, D = S.shape
    OUT = relation_kernel.shape[-1]
    BN = 2000

    def mm(s_ref, w_ref, o_ref):
        acc = jnp.dot(s_ref[0], w_ref[0], preferred_element_type=jnp.float32)
        for r in range(1, R):
            acc += jnp.dot(s_ref[r], w_ref[r],
                           preferred_element_type=jnp.float32)
        o_ref[...] = acc

    return pl.pallas_call(
        mm,
        grid=(N // BN,),
        in_specs=[
            pl.BlockSpec((R, BN, D), lambda i: (0, i, 0)),
            pl.BlockSpec((R, D, OUT), lambda i: (0, 0, 0)),
        ],
        out_specs=pl.BlockSpec((BN, OUT), lambda i: (i, 0)),
        out_shape=jax.ShapeDtypeStruct((N, OUT), jnp.float32),
    )(S, relation_kernel)


def _final_tc(head_e, tail_e, gath_ht, self_kernel):
    B, D = head_e.shape
    OUT = self_kernel.shape[-1]
    BB = 2048
    nblk = B // BB

    def fin(he, te, gh, gt, sk, oh, ot):
        oh[...] = jax.nn.sigmoid(
            jnp.dot(he[...], sk[...], preferred_element_type=jnp.float32)
            + gh[...])
        ot[...] = jax.nn.sigmoid(
            jnp.dot(te[...], sk[...], preferred_element_type=jnp.float32)
            + gt[...])

    return pl.pallas_call(
        fin,
        grid=(nblk,),
        in_specs=[
            pl.BlockSpec((BB, D), lambda i: (i, 0)),
            pl.BlockSpec((BB, D), lambda i: (i, 0)),
            pl.BlockSpec((BB, OUT), lambda i: (i, 0)),
            pl.BlockSpec((BB, OUT), lambda i: (i + nblk, 0)),
            pl.BlockSpec((D, OUT), lambda i: (0, 0)),
        ],
        out_specs=(pl.BlockSpec((BB, OUT), lambda i: (i, 0)),
                   pl.BlockSpec((BB, OUT), lambda i: (i, 0))),
        out_shape=(jax.ShapeDtypeStruct((B, OUT), jnp.float32),
                   jax.ShapeDtypeStruct((B, OUT), jnp.float32)),
    )(head_e, tail_e, gath_ht, gath_ht, self_kernel)


def kernel(embeddings, head_idx, head_e, tail_idx, tail_e, adj_src, adj_dst,
           relation_kernel, self_kernel):
    N, D = embeddings.shape
    R, E = adj_src.shape
    zeros = jnp.zeros((N - (NS - 1) * ((N // NS) // 8 * 8), D), jnp.float32)
    S = _seg_sum_sc(embeddings,
                    adj_src.reshape(R, E // CHUNK, CHUNK),
                    adj_dst.reshape(R, E // CHUNK, CHUNK), zeros)
    node_out = _relation_matmul_tc(S, relation_kernel)
    ht_idx = jnp.concatenate([head_idx, tail_idx]).reshape(-1, CHUNK)
    gath_ht = _gather_sc(node_out, ht_idx)
    return _final_tc(head_e, tail_e, gath_ht, self_kernel)


# speedup vs baseline: 1.2904x; 1.0121x over previous
import functools

import jax
import jax.numpy as jnp
from jax import lax
from jax.experimental import pallas as pl
from jax.experimental.pallas import tpu as pltpu
from jax.experimental.pallas import tpu_sc as plsc

NC = 2
NS = 16
NW = NC * NS

CHUNK = 128


def _seg_sum_sc(embeddings, adj_src, adj_dst, zeros):
    N, D = embeddings.shape
    R, E = adj_src.shape
    NCHT = E // CHUNK
    rel_per_core = R // NC
    npt = (N // NS) // 8 * 8
    npt_last = N - (NS - 1) * npt
    NCH = -(-NCHT // NS)
    NCH_LAST = NCHT - (NS - 1) * NCH
    assert 0 < NCH_LAST <= NCH
    NRB = 3
    NIB = 4

    mesh = plsc.VectorSubcoreMesh(core_axis_name="c", subcore_axis_name="s")

    @functools.partial(
        pl.kernel,
        out_type=jax.ShapeDtypeStruct((R, N, D), jnp.float32),
        mesh=mesh,
        scratch_types=[
            pltpu.VMEM((NIB, CHUNK), jnp.int32),
            pltpu.VMEM((NIB, CHUNK), jnp.int32),
            pltpu.VMEM((NRB, CHUNK, D), jnp.float32),
            pltpu.VMEM_SHARED((N, D), jnp.float32),
            pltpu.SemaphoreType.DMA,
            pltpu.SemaphoreType.DMA,
            pltpu.SemaphoreType.DMA,
        ],
    )
    def seg_sum(emb, asrc, adst, zer, out, isrc, idst, rows, acc,
                sem_i, sem_g, sem_s):
        c = lax.axis_index("c")
        s = lax.axis_index("s")

        def fire_idx(r, base, j):
            b = lax.rem(j, NIB)
            off = (base + j) * CHUNK
            pltpu.async_copy(asrc.at[r, pl.ds(off, CHUNK)], isrc.at[b],
                             sem_i)
            pltpu.async_copy(adst.at[r, pl.ds(off, CHUNK)], idst.at[b],
                             sem_i)

        def wait_idx():
            pltpu.make_async_copy(asrc.at[0, pl.ds(0, CHUNK)], isrc.at[0],
                                  sem_i).wait()
            pltpu.make_async_copy(adst.at[0, pl.ds(0, CHUNK)], idst.at[0],
                                  sem_i).wait()

        def fire_gather(j):
            pltpu.async_copy(emb.at[isrc.at[lax.rem(j, NIB)]],
                             rows.at[lax.rem(j, NRB)], sem_g)

        def wait_gather():
            pltpu.make_async_copy(emb.at[isrc.at[0]], rows.at[0],
                                  sem_g).wait()

        def fire_scatter(j):
            pltpu.async_copy(rows.at[lax.rem(j, NRB)],
                             acc.at[idst.at[lax.rem(j, NIB)]], sem_s,
                             add=True)

        def wait_scatter():
            pltpu.make_async_copy(rows.at[0], acc.at[pl.ds(0, CHUNK)],
                                  sem_s).wait()

        for rloc in range(rel_per_core):
            r = c * rel_per_core + rloc
            base = s * NCH
            nch_t = jnp.where(s < NS - 1, NCH, NCH_LAST)

            fire_idx(r, base, 0)
            fire_idx(r, base, 1)

            @pl.when(s < NS - 1)
            def _():
                pltpu.sync_copy(zer.at[pl.ds(0, npt)],
                                acc.at[pl.ds(s * npt, npt)])

            @pl.when(s == NS - 1)
            def _():
                pltpu.sync_copy(zer.at[pl.ds(0, npt_last)],
                                acc.at[pl.ds((NS - 1) * npt, npt_last)])

            wait_idx()
            fire_gather(0)
            plsc.subcore_barrier()

            @pl.loop(0, nch_t)
            def _(j):
                @pl.when(j >= 2)
                def _():
                    wait_scatter()

                @pl.when(j + 2 < nch_t)
                def _():
                    fire_idx(r, base, j + 2)

                @pl.when(j + 1 < nch_t)
                def _():
                    wait_idx()
                    fire_gather(j + 1)

                wait_gather()
                fire_scatter(j)

            wait_scatter()

            @pl.when(nch_t >= 2)
            def _():
                wait_scatter()

            plsc.subcore_barrier()

            @pl.when(s < NS - 1)
            def _():
                pltpu.sync_copy(acc.at[pl.ds(s * npt, npt)],
                                out.at[r, pl.ds(s * npt, npt)])

            @pl.when(s == NS - 1)
            def _():
                pltpu.sync_copy(acc.at[pl.ds((NS - 1) * npt, npt_last)],
                                out.at[r, pl.ds((NS - 1) * npt, npt_last)])

            plsc.subcore_barrier()

    return seg_sum(embeddings, adj_src, adj_dst, zeros)


def _gather_sc(node_out, ht_idx):
    N, D = node_out.shape
    ncht = ht_idx.shape[0] // CHUNK
    B = ncht * CHUNK // 2
    nch = ncht // NW
    assert nch * NW == ncht

    mesh = plsc.VectorSubcoreMesh(core_axis_name="c", subcore_axis_name="s")
    NB = 4

    @functools.partial(
        pl.kernel,
        out_type=jax.ShapeDtypeStruct((2 * B, D), jnp.float32),
        mesh=mesh,
        scratch_types=[
            pltpu.VMEM((nch * CHUNK,), jnp.int32),
            pltpu.VMEM((NB, CHUNK, D), jnp.float32),
            pltpu.SemaphoreType.DMA,
            pltpu.SemaphoreType.DMA,
        ],
    )
    def gat(node, htidx, o, idx_v, rows, sem, sem_st):
        c = lax.axis_index("c")
        s = lax.axis_index("s")
        w = s * NC + c

        pltpu.sync_copy(htidx.at[pl.ds(w * nch * CHUNK, nch * CHUNK)],
                        idx_v)

        def idx_chunk(j):
            return idx_v.at[pl.ds(j * CHUNK, CHUNK)]

        gd = [None] * nch
        std = [None] * nch
        for k in range(min(NB - 1, nch)):
            gd[k] = pltpu.async_copy(node.at[idx_chunk(k)], rows.at[k], sem)
        for j in range(nch):
            gd[j].wait()
            std[j] = pltpu.async_copy(
                rows.at[j % NB],
                o.at[pl.ds((w * nch + j) * CHUNK, CHUNK)], sem_st)
            nxt = j + NB - 1
            if nxt < nch:
                if j >= 1:
                    std[j - 1].wait()
                    std[j - 1] = None
                gd[nxt] = pltpu.async_copy(
                    node.at[idx_chunk(nxt)], rows.at[nxt % NB], sem)
        for j in range(nch):
            if std[j] is not None:
                std[j].wait()

    return gat(node_out, ht_idx)


def _relation_matmul_tc(S, relation_kernel):
    R, N, D = S.shape
    OUT = relation_kernel.shape[-1]
    BN = 2000

    def mm(s_ref, w_ref, o_ref):
        acc = jnp.dot(s_ref[0], w_ref[0], preferred_element_type=jnp.float32)
        for r in range(1, R):
            acc += jnp.dot(s_ref[r], w_ref[r],
                           preferred_element_type=jnp.float32)
        o_ref[...] = acc

    return pl.pallas_call(
        mm,
        grid=(N // BN,),
        in_specs=[
            pl.BlockSpec((R, BN, D), lambda i: (0, i, 0)),
            pl.BlockSpec((R, D, OUT), lambda i: (0, 0, 0)),
        ],
        out_specs=pl.BlockSpec((BN, OUT), lambda i: (i, 0)),
        out_shape=jax.ShapeDtypeStruct((N, OUT), jnp.float32),
    )(S, relation_kernel)


def _final_tc(head_e, tail_e, gath_ht, self_kernel):
    B, D = head_e.shape
    OUT = self_kernel.shape[-1]
    BB = 2048
    nblk = B // BB

    def fin(he, te, gh, gt, sk, oh, ot):
        oh[...] = jax.nn.sigmoid(
            jnp.dot(he[...], sk[...], preferred_element_type=jnp.float32)
            + gh[...])
        ot[...] = jax.nn.sigmoid(
            jnp.dot(te[...], sk[...], preferred_element_type=jnp.float32)
            + gt[...])

    return pl.pallas_call(
        fin,
        grid=(nblk,),
        in_specs=[
            pl.BlockSpec((BB, D), lambda i: (i, 0)),
            pl.BlockSpec((BB, D), lambda i: (i, 0)),
            pl.BlockSpec((BB, OUT), lambda i: (i, 0)),
            pl.BlockSpec((BB, OUT), lambda i: (i + nblk, 0)),
            pl.BlockSpec((D, OUT), lambda i: (0, 0)),
        ],
        out_specs=(pl.BlockSpec((BB, OUT), lambda i: (i, 0)),
                   pl.BlockSpec((BB, OUT), lambda i: (i, 0))),
        out_shape=(jax.ShapeDtypeStruct((B, OUT), jnp.float32),
                   jax.ShapeDtypeStruct((B, OUT), jnp.float32)),
    )(head_e, tail_e, gath_ht, gath_ht, self_kernel)


def kernel(embeddings, head_idx, head_e, tail_idx, tail_e, adj_src, adj_dst,
           relation_kernel, self_kernel):
    N, D = embeddings.shape
    R, E = adj_src.shape
    zeros = jnp.zeros((N - (NS - 1) * ((N // NS) // 8 * 8), D), jnp.float32)
    S = _seg_sum_sc(embeddings, adj_src, adj_dst, zeros)
    node_out = _relation_matmul_tc(S, relation_kernel)
    ht_idx = jnp.concatenate([head_idx, tail_idx])
    gath_ht = _gather_sc(node_out, ht_idx)
    return _final_tc(head_e, tail_e, gath_ht, self_kernel)


# speedup vs baseline: 1.3457x; 1.0429x over previous
import functools

import jax
import jax.numpy as jnp
from jax import lax
from jax.experimental import pallas as pl
from jax.experimental.pallas import tpu as pltpu
from jax.experimental.pallas import tpu_sc as plsc

NC = 2
NS = 16
NW = NC * NS

CHUNK = 128


def _seg_sum_sc(embeddings, adj_src, adj_dst, zeros):
    N, D = embeddings.shape
    R, E = adj_src.shape
    NCHT = E // CHUNK
    rel_per_core = R // NC
    npt = (N // NS) // 8 * 8
    npt_last = N - (NS - 1) * npt
    NCH = -(-NCHT // NS)
    NCH_LAST = NCHT - (NS - 1) * NCH
    assert 0 < NCH_LAST <= NCH
    NRB = 3
    NIB = 4

    mesh = plsc.VectorSubcoreMesh(core_axis_name="c", subcore_axis_name="s")

    @functools.partial(
        pl.kernel,
        out_type=jax.ShapeDtypeStruct((R, N, D), jnp.float32),
        mesh=mesh,
        scratch_types=[
            pltpu.VMEM((NIB, CHUNK), jnp.int32),
            pltpu.VMEM((NIB, CHUNK), jnp.int32),
            pltpu.VMEM((NRB, CHUNK, D), jnp.float32),
            pltpu.VMEM_SHARED((N, D), jnp.float32),
            pltpu.SemaphoreType.DMA,
            pltpu.SemaphoreType.DMA,
            pltpu.SemaphoreType.DMA,
        ],
    )
    def seg_sum(emb, asrc, adst, zer, out, isrc, idst, rows, acc,
                sem_i, sem_g, sem_s):
        c = lax.axis_index("c")
        s = lax.axis_index("s")

        def fire_idx(r, base, j):
            b = lax.rem(j, NIB)
            off = (base + j) * CHUNK
            pltpu.async_copy(asrc.at[r, pl.ds(off, CHUNK)], isrc.at[b],
                             sem_i)
            pltpu.async_copy(adst.at[r, pl.ds(off, CHUNK)], idst.at[b],
                             sem_i)

        def wait_idx():
            pltpu.make_async_copy(asrc.at[0, pl.ds(0, CHUNK)], isrc.at[0],
                                  sem_i).wait()
            pltpu.make_async_copy(adst.at[0, pl.ds(0, CHUNK)], idst.at[0],
                                  sem_i).wait()

        def fire_gather(j):
            pltpu.async_copy(emb.at[isrc.at[lax.rem(j, NIB)]],
                             rows.at[lax.rem(j, NRB)], sem_g)

        def wait_gather():
            pltpu.make_async_copy(emb.at[isrc.at[0]], rows.at[0],
                                  sem_g).wait()

        def fire_scatter(j):
            pltpu.async_copy(rows.at[lax.rem(j, NRB)],
                             acc.at[idst.at[lax.rem(j, NIB)]], sem_s,
                             add=True)

        def wait_scatter():
            pltpu.make_async_copy(rows.at[0], acc.at[pl.ds(0, CHUNK)],
                                  sem_s).wait()

        for rloc in range(rel_per_core):
            r = c * rel_per_core + rloc
            base = s * NCH
            nch_t = jnp.where(s < NS - 1, NCH, NCH_LAST)

            fire_idx(r, base, 0)
            fire_idx(r, base, 1)

            if rloc == 0:
                @pl.when(s < NS - 1)
                def _():
                    pltpu.sync_copy(zer.at[pl.ds(0, npt)],
                                    acc.at[pl.ds(s * npt, npt)])

                @pl.when(s == NS - 1)
                def _():
                    pltpu.sync_copy(zer.at[pl.ds(0, npt_last)],
                                    acc.at[pl.ds((NS - 1) * npt, npt_last)])

            wait_idx()
            fire_gather(0)
            plsc.subcore_barrier()

            @pl.loop(0, nch_t)
            def _(j):
                @pl.when(j >= 2)
                def _():
                    wait_scatter()

                @pl.when(j + 2 < nch_t)
                def _():
                    fire_idx(r, base, j + 2)

                @pl.when(j + 1 < nch_t)
                def _():
                    wait_idx()
                    fire_gather(j + 1)

                wait_gather()
                fire_scatter(j)

            wait_scatter()

            @pl.when(nch_t >= 2)
            def _():
                wait_scatter()

            plsc.subcore_barrier()

            @pl.when(s < NS - 1)
            def _():
                pltpu.sync_copy(acc.at[pl.ds(s * npt, npt)],
                                out.at[r, pl.ds(s * npt, npt)])

            @pl.when(s == NS - 1)
            def _():
                pltpu.sync_copy(acc.at[pl.ds((NS - 1) * npt, npt_last)],
                                out.at[r, pl.ds((NS - 1) * npt, npt_last)])

            plsc.subcore_barrier()

    return seg_sum(embeddings, adj_src, adj_dst, zeros)


def _gather_sc(node_out, ht_idx):
    N, D = node_out.shape
    dt = node_out.dtype
    ncht = ht_idx.shape[0] // CHUNK
    B = ncht * CHUNK // 2
    nch = ncht // NW
    assert nch * NW == ncht

    mesh = plsc.VectorSubcoreMesh(core_axis_name="c", subcore_axis_name="s")
    NB = 4

    @functools.partial(
        pl.kernel,
        out_type=jax.ShapeDtypeStruct((2 * B, D), dt),
        mesh=mesh,
        scratch_types=[
            pltpu.VMEM((nch * CHUNK,), jnp.int32),
            pltpu.VMEM((NB, CHUNK, D), dt),
            pltpu.SemaphoreType.DMA,
            pltpu.SemaphoreType.DMA,
        ],
    )
    def gat(node, htidx, o, idx_v, rows, sem, sem_st):
        c = lax.axis_index("c")
        s = lax.axis_index("s")
        w = s * NC + c

        pltpu.sync_copy(htidx.at[pl.ds(w * nch * CHUNK, nch * CHUNK)],
                        idx_v)

        def idx_chunk(j):
            return idx_v.at[pl.ds(j * CHUNK, CHUNK)]

        gd = [None] * nch
        std = [None] * nch
        for k in range(min(NB - 1, nch)):
            gd[k] = pltpu.async_copy(node.at[idx_chunk(k)], rows.at[k], sem)
        for j in range(nch):
            gd[j].wait()
            std[j] = pltpu.async_copy(
                rows.at[j % NB],
                o.at[pl.ds((w * nch + j) * CHUNK, CHUNK)], sem_st)
            nxt = j + NB - 1
            if nxt < nch:
                if j >= 1:
                    std[j - 1].wait()
                    std[j - 1] = None
                gd[nxt] = pltpu.async_copy(
                    node.at[idx_chunk(nxt)], rows.at[nxt % NB], sem)
        for j in range(nch):
            if std[j] is not None:
                std[j].wait()

    return gat(node_out, ht_idx)


def _relation_matmul_tc(S, relation_kernel):
    R, N, D = S.shape
    OUT = relation_kernel.shape[-1]
    BN = 2000

    def mm(s_ref, w_ref, o_ref):
        acc = jnp.dot(s_ref[0], w_ref[0], preferred_element_type=jnp.float32)
        for r in range(1, R):
            acc += jnp.dot(s_ref[r], w_ref[r],
                           preferred_element_type=jnp.float32)
        o_ref[...] = acc

    return pl.pallas_call(
        mm,
        grid=(N // BN,),
        in_specs=[
            pl.BlockSpec((R, BN, D), lambda i: (0, i, 0)),
            pl.BlockSpec((R, D, OUT), lambda i: (0, 0, 0)),
        ],
        out_specs=pl.BlockSpec((BN, OUT), lambda i: (i, 0)),
        out_shape=jax.ShapeDtypeStruct((N, OUT), jnp.float32),
    )(S, relation_kernel)


def _final_tc(head_e, tail_e, gath_ht, self_kernel):
    B, D = head_e.shape
    OUT = self_kernel.shape[-1]
    BB = 2048
    nblk = B // BB

    def fin(he, te, gh, gt, sk, oh, ot):
        oh[...] = jax.nn.sigmoid(
            jnp.dot(he[...], sk[...], preferred_element_type=jnp.float32)
            + gh[...])
        ot[...] = jax.nn.sigmoid(
            jnp.dot(te[...], sk[...], preferred_element_type=jnp.float32)
            + gt[...])

    return pl.pallas_call(
        fin,
        grid=(nblk,),
        in_specs=[
            pl.BlockSpec((BB, D), lambda i: (i, 0)),
            pl.BlockSpec((BB, D), lambda i: (i, 0)),
            pl.BlockSpec((BB, OUT), lambda i: (i, 0)),
            pl.BlockSpec((BB, OUT), lambda i: (i + nblk, 0)),
            pl.BlockSpec((D, OUT), lambda i: (0, 0)),
        ],
        out_specs=(pl.BlockSpec((BB, OUT), lambda i: (i, 0)),
                   pl.BlockSpec((BB, OUT), lambda i: (i, 0))),
        out_shape=(jax.ShapeDtypeStruct((B, OUT), jnp.float32),
                   jax.ShapeDtypeStruct((B, OUT), jnp.float32)),
    )(head_e, tail_e, gath_ht, gath_ht, self_kernel)


def kernel(embeddings, head_idx, head_e, tail_idx, tail_e, adj_src, adj_dst,
           relation_kernel, self_kernel):
    N, D = embeddings.shape
    R, E = adj_src.shape
    zeros = jnp.zeros((N - (NS - 1) * ((N // NS) // 8 * 8), D), jnp.float32)
    rel_per_core = R // NC
    w_next = jnp.concatenate(
        [relation_kernel[1:], jnp.zeros_like(relation_kernel[:1])])
    keep = jnp.asarray(
        [(i % rel_per_core) != rel_per_core - 1 for i in range(R)],
        jnp.float32)[:, None, None]
    w_eff = relation_kernel - w_next * keep
    S = _seg_sum_sc(embeddings, adj_src, adj_dst, zeros)
    node_out = _relation_matmul_tc(S, w_eff)
    ht_idx = jnp.concatenate([head_idx, tail_idx])
    gath_ht = _gather_sc(node_out, ht_idx)
    return _final_tc(head_e, tail_e, gath_ht, self_kernel)
